# baseline (speedup 1.0000x reference)
import jax
import jax.numpy as jnp
from jax import lax
from jax.experimental import pallas as pl
from jax.experimental.pallas import tpu as pltpu
from jax.experimental.pallas import tpu_sc as plsc

_EMB = 64
_SEQ = 200
_NTAGS_PAD = 1024
_NCHUNKS = _SEQ // 8


def _cbow_body(words_hbm, table_hbm, wblk_hbm, biasp_hbm, out_hbm,
               idx_a, idx_b, rows_a, rows_b, part_v, shared_sp, sblk_v,
               wv, bias_v, res_v, gsem, wsem, bsem):
    c = lax.axis_index("c")
    s = lax.axis_index("s")
    wid = c * 16 + s

    wcp = pltpu.async_copy(wblk_hbm.at[wid], wv, wsem)
    bcp = pltpu.async_copy(biasp_hbm.at[pl.ds(wid * 32, 32)], bias_v, bsem)

    pltpu.sync_copy(words_hbm.at[pl.ds(s * 8, 8)], idx_a)
    pltpu.async_copy(table_hbm.at[idx_a], rows_a, gsem).wait()
    for g in range(4):
        acc = rows_a[0, pl.ds(g * 16, 16)]
        for r in range(1, 8):
            acc = acc + rows_a[r, pl.ds(g * 16, 16)]
        part_v[pl.ds(g * 16, 16)] = acc

    @pl.when(s < _NCHUNKS - 16)
    def _second_chunk():
        pltpu.sync_copy(words_hbm.at[pl.ds(128 + s * 8, 8)], idx_b)
        pltpu.async_copy(table_hbm.at[idx_b], rows_b, gsem).wait()
        for g in range(4):
            acc = part_v[pl.ds(g * 16, 16)]
            for r in range(8):
                acc = acc + rows_b[r, pl.ds(g * 16, 16)]
            part_v[pl.ds(g * 16, 16)] = acc

    pltpu.sync_copy(part_v, shared_sp.at[s])
    plsc.subcore_barrier()
    pltpu.sync_copy(shared_sp, sblk_v)
    svecs = []
    for g in range(4):
        acc = sblk_v[0, pl.ds(g * 16, 16)]
        for r in range(1, 16):
            acc = acc + sblk_v[r, pl.ds(g * 16, 16)]
        svecs.append(acc)

    wcp.wait()
    bcp.wait()
    acc0 = bias_v[pl.ds(0, 16)]
    acc1 = bias_v[pl.ds(16, 16)]
    for e in range(_EMB):
        se = svecs[e // 16][e % 16]
        acc0 = acc0 + se * wv[e, pl.ds(0, 16)]
        acc1 = acc1 + se * wv[e, pl.ds(16, 16)]
    res_v[pl.ds(0, 16)] = acc0
    res_v[pl.ds(16, 16)] = acc1
    pltpu.sync_copy(res_v, out_hbm.at[pl.ds(wid * 32, 32)])


_mesh = plsc.VectorSubcoreMesh(core_axis_name="c", subcore_axis_name="s",
                               num_cores=2, num_subcores=16)

_cbow_call = pl.kernel(
    _cbow_body,
    out_type=jax.ShapeDtypeStruct((_NTAGS_PAD,), jnp.float32),
    mesh=_mesh,
    scratch_types=[
        pltpu.VMEM((8,), jnp.int32),
        pltpu.VMEM((8,), jnp.int32),
        pltpu.VMEM((8, _EMB), jnp.float32),
        pltpu.VMEM((8, _EMB), jnp.float32),
        pltpu.VMEM((_EMB,), jnp.float32),
        pltpu.VMEM_SHARED((16, _EMB), jnp.float32),
        pltpu.VMEM((16, _EMB), jnp.float32),
        pltpu.VMEM((_EMB, 32), jnp.float32),
        pltpu.VMEM((32,), jnp.float32),
        pltpu.VMEM((32,), jnp.float32),
        pltpu.SemaphoreType.DMA,
        pltpu.SemaphoreType.DMA,
        pltpu.SemaphoreType.DMA,
    ],
    compiler_params=pltpu.CompilerParams(use_tc_tiling_on_sc=False),
)


@jax.jit
def kernel(words, emb_table, W, bias):
    words = words.astype(jnp.int32)
    wpad = jnp.pad(W, ((0, _NTAGS_PAD - W.shape[0]), (0, 0)))
    wblk = wpad.reshape(32, 32, _EMB).transpose(0, 2, 1)
    biasp = jnp.pad(bias.reshape(-1), (0, _NTAGS_PAD - bias.size))
    out = _cbow_call(words, emb_table, wblk, biasp)
    return out[: bias.size].reshape(1, -1)


# speedup vs baseline: 18.7798x; 18.7798x over previous
import jax
import jax.numpy as jnp
from jax import lax
from jax.experimental import pallas as pl
from jax.experimental.pallas import tpu as pltpu
from jax.experimental.pallas import tpu_sc as plsc

_EMB = 64
_SEQ = 200
_NTAGS_PAD = 1024
_NCHUNKS = _SEQ // 8


def _cbow_body(words_hbm, tablet_hbm, wblk_hbm, biasp_hbm, out_hbm,
               words_v, tile0_v, tile1_v, part_v, shared_sp, sblk_v,
               wv, bias_v, res_v, tsem0, tsem1, wsem, bsem):
    c = lax.axis_index("c")
    s = lax.axis_index("s")
    wid = c * 16 + s
    iota16 = lax.iota(jnp.int32, 16)

    wcp = pltpu.async_copy(wblk_hbm.at[wid], wv, wsem)
    bcp = pltpu.async_copy(biasp_hbm.at[pl.ds(wid * 32, 32)], bias_v, bsem)

    pltpu.sync_copy(words_hbm.at[pl.ds(s * 8, 8)], words_v.at[pl.ds(0, 8)])

    @pl.when(s < _NCHUNKS - 16)
    def _load_second_chunk():
        pltpu.sync_copy(words_hbm.at[pl.ds(128 + s * 8, 8)],
                        words_v.at[pl.ds(8, 8)])

    wvec = words_v[...]
    tiles = [tile0_v, tile1_v]
    sems = [tsem0, tsem1]

    def issue(k):
        w = wvec[k]
        col0 = pl.multiple_of((w >> 7) * 128, 128)
        return pltpu.async_copy(tablet_hbm.at[:, pl.ds(col0, 128)],
                                tiles[k % 2], sems[k % 2])

    def extract(k, accs):
        lane = jnp.full((16,), wvec[k] & 127, jnp.int32)
        tile = tiles[k % 2]
        return [acc + plsc.load_gather(tile, [iota16 + g * 16, lane])
                for g, acc in enumerate(accs)]

    accs = [jnp.zeros((16,), jnp.float32) for _ in range(4)]
    cp = issue(0)
    for k in range(1, 8):
        cp_next = issue(k)
        cp.wait()
        accs = extract(k - 1, accs)
        cp = cp_next
    cp.wait()
    accs = extract(7, accs)
    for g in range(4):
        part_v[pl.ds(g * 16, 16)] = accs[g]

    @pl.when(s < _NCHUNKS - 16)
    def _second_chunk():
        accs2 = [part_v[pl.ds(g * 16, 16)] for g in range(4)]
        cp2 = issue(8)
        for k in range(9, 16):
            cp2_next = issue(k)
            cp2.wait()
            accs2 = extract(k - 1, accs2)
            cp2 = cp2_next
        cp2.wait()
        accs2 = extract(15, accs2)
        for g in range(4):
            part_v[pl.ds(g * 16, 16)] = accs2[g]

    pltpu.sync_copy(part_v, shared_sp.at[s])
    plsc.subcore_barrier()
    pltpu.sync_copy(shared_sp, sblk_v)
    svecs = []
    for g in range(4):
        acc = sblk_v[0, pl.ds(g * 16, 16)]
        for r in range(1, 16):
            acc = acc + sblk_v[r, pl.ds(g * 16, 16)]
        svecs.append(acc)

    wcp.wait()
    bcp.wait()
    acc0 = bias_v[pl.ds(0, 16)]
    acc1 = bias_v[pl.ds(16, 16)]
    for e in range(_EMB):
        se = svecs[e // 16][e % 16]
        acc0 = acc0 + se * wv[e, pl.ds(0, 16)]
        acc1 = acc1 + se * wv[e, pl.ds(16, 16)]
    res_v[pl.ds(0, 16)] = acc0
    res_v[pl.ds(16, 16)] = acc1
    pltpu.sync_copy(res_v, out_hbm.at[pl.ds(wid * 32, 32)])


_mesh = plsc.VectorSubcoreMesh(core_axis_name="c", subcore_axis_name="s",
                               num_cores=2, num_subcores=16)

_cbow_call = pl.kernel(
    _cbow_body,
    out_type=jax.ShapeDtypeStruct((_NTAGS_PAD,), jnp.float32),
    mesh=_mesh,
    scratch_types=[
        pltpu.VMEM((16,), jnp.int32),
        pltpu.VMEM((_EMB, 128), jnp.float32),
        pltpu.VMEM((_EMB, 128), jnp.float32),
        pltpu.VMEM((128,), jnp.float32),
        pltpu.VMEM_SHARED((16, 128), jnp.float32),
        pltpu.VMEM((16, 128), jnp.float32),
        pltpu.VMEM((_EMB, 128), jnp.float32),
        pltpu.VMEM((32,), jnp.float32),
        pltpu.VMEM((32,), jnp.float32),
        pltpu.SemaphoreType.DMA,
        pltpu.SemaphoreType.DMA,
        pltpu.SemaphoreType.DMA,
        pltpu.SemaphoreType.DMA,
    ],
    compiler_params=pltpu.CompilerParams(use_tc_tiling_on_sc=True,
                                         needs_layout_passes=False),
)


@jax.jit
def kernel(words, emb_table, W, bias):
    words = words.astype(jnp.int32)
    tablet = emb_table.T
    wpad = jnp.pad(W, ((0, _NTAGS_PAD - W.shape[0]), (0, 0)))
    wblk = jnp.pad(wpad.reshape(32, 32, _EMB).transpose(0, 2, 1),
                   ((0, 0), (0, 0), (0, 96)))
    biasp = jnp.pad(bias.reshape(-1), (0, _NTAGS_PAD - bias.size))
    out = _cbow_call(words, tablet, wblk, biasp)
    return out[: bias.size].reshape(1, -1)


# speedup vs baseline: 21.0111x; 1.1188x over previous
import jax
import jax.numpy as jnp
from jax import lax
from jax.experimental import pallas as pl
from jax.experimental.pallas import tpu as pltpu
from jax.experimental.pallas import tpu_sc as plsc

_EMB = 64
_SEQ = 200
_NTAGS_PAD = 1024
_NCHUNKS = _SEQ // 8
_NBUF = 4


def _cbow_body(words_hbm, tablet_hbm, wblk_hbm, biasp_hbm, out_hbm,
               words_v, tiles_v, part_v, shared_sp, sblk_v,
               wv, bias_v, res_v, tsems, wsem, bsem):
    s = lax.axis_index("s")
    iota16 = lax.iota(jnp.int32, 16)

    wcp = pltpu.async_copy(wblk_hbm.at[s], wv, wsem)
    bcp = pltpu.async_copy(biasp_hbm.at[pl.ds(s * 64, 64)], bias_v, bsem)

    pltpu.sync_copy(words_hbm.at[pl.ds(s * 8, 8)], words_v.at[pl.ds(0, 8)])

    @pl.when(s < _NCHUNKS - 16)
    def _load_second_chunk():
        pltpu.sync_copy(words_hbm.at[pl.ds(128 + s * 8, 8)],
                        words_v.at[pl.ds(8, 8)])

    wvec = words_v[...]

    def issue(k):
        w = wvec[k]
        col0 = pl.multiple_of((w >> 7) * 128, 128)
        return pltpu.async_copy(tablet_hbm.at[:, pl.ds(col0, 128)],
                                tiles_v.at[k % _NBUF], tsems[k % _NBUF])

    def extract(k, accs):
        lane = jnp.full((16,), wvec[k] & 127, jnp.int32)
        buf = jnp.full((16,), k % _NBUF, jnp.int32)
        return [acc + plsc.load_gather(tiles_v, [buf, iota16 + g * 16, lane])
                for g, acc in enumerate(accs)]

    accs = [jnp.zeros((16,), jnp.float32) for _ in range(4)]
    cps = [issue(k) for k in range(_NBUF - 1)]
    for k in range(_NBUF - 1, 8):
        cps.append(issue(k))
        cps[k - (_NBUF - 1)].wait()
        accs = extract(k - (_NBUF - 1), accs)
    for k in range(8 - (_NBUF - 1), 8):
        cps[k].wait()
        accs = extract(k, accs)
    for g in range(4):
        part_v[pl.ds(g * 16, 16)] = accs[g]

    @pl.when(s < _NCHUNKS - 16)
    def _second_chunk():
        accs2 = [part_v[pl.ds(g * 16, 16)] for g in range(4)]
        cps2 = [issue(k) for k in range(8, 8 + _NBUF - 1)]
        for k in range(8 + _NBUF - 1, 16):
            cps2.append(issue(k))
            cps2[k - 8 - (_NBUF - 1)].wait()
            accs2 = extract(k - (_NBUF - 1), accs2)
        for k in range(16 - (_NBUF - 1), 16):
            cps2[k - 8].wait()
            accs2 = extract(k, accs2)
        for g in range(4):
            part_v[pl.ds(g * 16, 16)] = accs2[g]

    pltpu.sync_copy(part_v, shared_sp.at[s])
    plsc.subcore_barrier()
    pltpu.sync_copy(shared_sp, sblk_v)
    svecs = []
    for g in range(4):
        acc = sblk_v[0, pl.ds(g * 16, 16)]
        for r in range(1, 16):
            acc = acc + sblk_v[r, pl.ds(g * 16, 16)]
        svecs.append(acc)

    wcp.wait()
    bcp.wait()
    accs_o = [bias_v[pl.ds(q * 16, 16)] for q in range(4)]
    for e in range(_EMB):
        se = svecs[e // 16][e % 16]
        accs_o = [acc + se * wv[e, pl.ds(q * 16, 16)]
                  for q, acc in enumerate(accs_o)]
    for q in range(4):
        res_v[pl.ds(q * 16, 16)] = accs_o[q]
    pltpu.sync_copy(res_v, out_hbm.at[pl.ds(s * 64, 64)])


_mesh = plsc.VectorSubcoreMesh(core_axis_name="c", subcore_axis_name="s",
                               num_cores=1, num_subcores=16)

_cbow_call = pl.kernel(
    _cbow_body,
    out_type=jax.ShapeDtypeStruct((_NTAGS_PAD,), jnp.float32),
    mesh=_mesh,
    scratch_types=[
        pltpu.VMEM((16,), jnp.int32),
        pltpu.VMEM((_NBUF, _EMB, 128), jnp.float32),
        pltpu.VMEM((128,), jnp.float32),
        pltpu.VMEM_SHARED((16, 128), jnp.float32),
        pltpu.VMEM((16, 128), jnp.float32),
        pltpu.VMEM((_EMB, 128), jnp.float32),
        pltpu.VMEM((64,), jnp.float32),
        pltpu.VMEM((64,), jnp.float32),
        [pltpu.SemaphoreType.DMA] * _NBUF,
        pltpu.SemaphoreType.DMA,
        pltpu.SemaphoreType.DMA,
    ],
    compiler_params=pltpu.CompilerParams(use_tc_tiling_on_sc=True,
                                         needs_layout_passes=False),
)


@jax.jit
def kernel(words, emb_table, W, bias):
    words = words.astype(jnp.int32)
    tablet = emb_table.T
    wpad = jnp.pad(W, ((0, _NTAGS_PAD - W.shape[0]), (0, 0)))
    wblk = jnp.pad(wpad.reshape(16, 64, _EMB).transpose(0, 2, 1),
                   ((0, 0), (0, 0), (0, 64)))
    biasp = jnp.pad(bias.reshape(-1), (0, _NTAGS_PAD - bias.size))
    out = _cbow_call(words, tablet, wblk, biasp)
    return out[: bias.size].reshape(1, -1)


# speedup vs baseline: 22.4055x; 1.0664x over previous
import jax
import jax.numpy as jnp
from jax import lax
from jax.experimental import pallas as pl
from jax.experimental.pallas import tpu as pltpu
from jax.experimental.pallas import tpu_sc as plsc

_EMB = 64
_SEQ = 200
_NTAGS_PAD = 1024
_NBUF = 4


def _cbow_body(words_hbm, tablet_hbm, wt_hbm, bias_hbm, out_hbm,
               words_v, tiles_v, part_v, shared_sp, sblk_v,
               wv, bias_v, res_v, tsems, wsem, bsem):
    s = lax.axis_index("s")
    iota16 = lax.iota(jnp.int32, 16)

    wcol0 = pl.multiple_of((s >> 1) * 128, 128)
    wcp = pltpu.async_copy(wt_hbm.at[:, pl.ds(wcol0, 128)], wv, wsem)
    bcp = pltpu.async_copy(bias_hbm.at[pl.ds(s * 64, 64)], bias_v, bsem)

    pltpu.sync_copy(words_hbm, words_v.at[pl.ds(0, _SEQ)])
    wvec = plsc.load_gather(words_v, [s + 16 * iota16])

    def issue(j):
        w = wvec[j]
        col0 = pl.multiple_of((w >> 7) * 128, 128)
        return pltpu.async_copy(tablet_hbm.at[:, pl.ds(col0, 128)],
                                tiles_v.at[j % _NBUF], tsems[j % _NBUF])

    def extract(j, accs):
        lane = jnp.full((16,), wvec[j] & 127, jnp.int32)
        buf = jnp.full((16,), j % _NBUF, jnp.int32)
        return [acc + plsc.load_gather(tiles_v, [buf, iota16 + g * 16, lane])
                for g, acc in enumerate(accs)]

    accs = [jnp.zeros((16,), jnp.float32) for _ in range(4)]
    cps = [issue(j) for j in range(_NBUF - 1)]
    for j in range(_NBUF - 1, 12):
        cps.append(issue(j))
        cps[j - (_NBUF - 1)].wait()
        accs = extract(j - (_NBUF - 1), accs)
    for j in range(12 - (_NBUF - 1), 12):
        cps[j].wait()
        accs = extract(j, accs)
    for g in range(4):
        part_v[pl.ds(g * 16, 16)] = accs[g]

    @pl.when(s < _SEQ - 192)
    def _thirteenth():
        cp = issue(12)
        cp.wait()
        accs2 = [part_v[pl.ds(g * 16, 16)] for g in range(4)]
        accs2 = extract(12, accs2)
        for g in range(4):
            part_v[pl.ds(g * 16, 16)] = accs2[g]

    pltpu.sync_copy(part_v, shared_sp.at[s])
    plsc.subcore_barrier()
    pltpu.sync_copy(shared_sp, sblk_v)
    svecs = []
    for g in range(4):
        acc = sblk_v[0, pl.ds(g * 16, 16)]
        for r in range(1, 16):
            acc = acc + sblk_v[r, pl.ds(g * 16, 16)]
        svecs.append(acc)

    wcp.wait()
    bcp.wait()
    lanes = [(s & 1) * 64 + q * 16 + iota16 for q in range(4)]
    accs_o = [bias_v[pl.ds(q * 16, 16)] for q in range(4)]
    for e in range(_EMB):
        se = svecs[e // 16][e % 16]
        erow = jnp.full((16,), e, jnp.int32)
        accs_o = [acc + se * plsc.load_gather(wv, [erow, lanes[q]])
                  for q, acc in enumerate(accs_o)]
    for q in range(4):
        res_v[pl.ds(q * 16, 16)] = accs_o[q]
    pltpu.sync_copy(res_v, out_hbm.at[pl.ds(s * 64, 64)])


_mesh = plsc.VectorSubcoreMesh(core_axis_name="c", subcore_axis_name="s",
                               num_cores=1, num_subcores=16)

_cbow_call = pl.kernel(
    _cbow_body,
    out_type=jax.ShapeDtypeStruct((_NTAGS_PAD,), jnp.float32),
    mesh=_mesh,
    scratch_types=[
        pltpu.VMEM((256,), jnp.int32),
        pltpu.VMEM((_NBUF, _EMB, 128), jnp.float32),
        pltpu.VMEM((128,), jnp.float32),
        pltpu.VMEM_SHARED((16, 128), jnp.float32),
        pltpu.VMEM((16, 128), jnp.float32),
        pltpu.VMEM((_EMB, 128), jnp.float32),
        pltpu.VMEM((64,), jnp.float32),
        pltpu.VMEM((64,), jnp.float32),
        [pltpu.SemaphoreType.DMA] * _NBUF,
        pltpu.SemaphoreType.DMA,
        pltpu.SemaphoreType.DMA,
    ],
    compiler_params=pltpu.CompilerParams(use_tc_tiling_on_sc=True,
                                         needs_layout_passes=False),
)


@jax.jit
def kernel(words, emb_table, W, bias):
    words = words.astype(jnp.int32)
    out = _cbow_call(words, emb_table.T, W.T, bias.reshape(-1))
    return out[: bias.size].reshape(1, -1)


# speedup vs baseline: 23.0832x; 1.0302x over previous
import jax
import jax.numpy as jnp
from jax import lax
from jax.experimental import pallas as pl
from jax.experimental.pallas import tpu as pltpu
from jax.experimental.pallas import tpu_sc as plsc

_EMB = 64
_SEQ = 200
_NTAGS_PAD = 1024
_NBUF = 6


def _cbow_body(words_hbm, tablet_hbm, wt_hbm, bias_hbm, out_hbm,
               words_v, tiles_v, part_v, shared_sp, sblk_v,
               wv, bias_v, res_v, tsems, wsem, bsem):
    s = lax.axis_index("s")
    iota16 = lax.iota(jnp.int32, 16)

    wcol0 = pl.multiple_of((s >> 1) * 128, 128)
    wcp = pltpu.async_copy(wt_hbm.at[:, pl.ds(wcol0, 128)], wv, wsem)
    bcp = pltpu.async_copy(bias_hbm.at[pl.ds(s * 64, 64)], bias_v, bsem)

    pltpu.sync_copy(words_hbm, words_v.at[pl.ds(0, _SEQ)])
    wvec = plsc.load_gather(words_v, [s + 16 * iota16])

    def issue(j):
        w = wvec[j]
        col0 = pl.multiple_of((w >> 7) * 128, 128)
        return pltpu.async_copy(tablet_hbm.at[:, pl.ds(col0, 128)],
                                tiles_v.at[j % _NBUF], tsems[j % _NBUF])

    def extract(j, accs):
        lane = jnp.full((16,), wvec[j] & 127, jnp.int32)
        buf = jnp.full((16,), j % _NBUF, jnp.int32)
        return [acc + plsc.load_gather(tiles_v, [buf, iota16 + g * 16, lane])
                for g, acc in enumerate(accs)]

    accs = [jnp.zeros((16,), jnp.float32) for _ in range(4)]
    cps = [issue(j) for j in range(_NBUF - 1)]
    for j in range(_NBUF - 1, 12):
        cps.append(issue(j))
        cps[j - (_NBUF - 1)].wait()
        accs = extract(j - (_NBUF - 1), accs)
    for j in range(12 - (_NBUF - 1), 12):
        cps[j].wait()
        accs = extract(j, accs)
    for g in range(4):
        part_v[pl.ds(g * 16, 16)] = accs[g]

    @pl.when(s < _SEQ - 192)
    def _thirteenth():
        cp = issue(12)
        cp.wait()
        accs2 = [part_v[pl.ds(g * 16, 16)] for g in range(4)]
        accs2 = extract(12, accs2)
        for g in range(4):
            part_v[pl.ds(g * 16, 16)] = accs2[g]

    pltpu.sync_copy(part_v, shared_sp.at[s])
    plsc.subcore_barrier()
    pltpu.sync_copy(shared_sp, sblk_v)
    svecs = []
    for g in range(4):
        acc = sblk_v[0, pl.ds(g * 16, 16)]
        for r in range(1, 16):
            acc = acc + sblk_v[r, pl.ds(g * 16, 16)]
        svecs.append(acc)

    wcp.wait()
    bcp.wait()
    lanes = [(s & 1) * 64 + q * 16 + iota16 for q in range(4)]
    accs_o = [bias_v[pl.ds(q * 16, 16)] for q in range(4)]
    for e in range(_EMB):
        se = svecs[e // 16][e % 16]
        erow = jnp.full((16,), e, jnp.int32)
        accs_o = [acc + se * plsc.load_gather(wv, [erow, lanes[q]])
                  for q, acc in enumerate(accs_o)]
    for q in range(4):
        res_v[pl.ds(q * 16, 16)] = accs_o[q]
    pltpu.sync_copy(res_v, out_hbm.at[pl.ds(s * 64, 64)])


_mesh = plsc.VectorSubcoreMesh(core_axis_name="c", subcore_axis_name="s",
                               num_cores=1, num_subcores=16)

_cbow_call = pl.kernel(
    _cbow_body,
    out_type=jax.ShapeDtypeStruct((_NTAGS_PAD,), jnp.float32),
    mesh=_mesh,
    scratch_types=[
        pltpu.VMEM((256,), jnp.int32),
        pltpu.VMEM((_NBUF, _EMB, 128), jnp.float32),
        pltpu.VMEM((128,), jnp.float32),
        pltpu.VMEM_SHARED((16, 128), jnp.float32),
        pltpu.VMEM((16, 128), jnp.float32),
        pltpu.VMEM((_EMB, 128), jnp.float32),
        pltpu.VMEM((64,), jnp.float32),
        pltpu.VMEM((64,), jnp.float32),
        [pltpu.SemaphoreType.DMA] * _NBUF,
        pltpu.SemaphoreType.DMA,
        pltpu.SemaphoreType.DMA,
    ],
    compiler_params=pltpu.CompilerParams(use_tc_tiling_on_sc=True,
                                         needs_layout_passes=False),
)


@jax.jit
def kernel(words, emb_table, W, bias):
    words = words.astype(jnp.int32)
    out = _cbow_call(words, emb_table.T, W.T, bias.reshape(-1))
    return out[: bias.size].reshape(1, -1)
